# baseline (speedup 1.0000x reference)
import functools

import numpy as np
import jax
import jax.numpy as jnp
from jax import lax
from jax.experimental import pallas as pl
from jax.experimental.pallas import tpu as pltpu
from jax.experimental.pallas import tpu_sc as plsc

N, D, H, E, G = 10000, 128, 128, 320000, 128
NB = 10
NC, NS = 2, 16
NW = NC * NS
EPW = E // NW
CHUNK = 128
NCHUNK = 80
HALF = NCHUNK // 2
EPAD = NCHUNK * CHUNK - EPW
NPAD = N + 16
DUMMY = N + 8
ROWS_PT = 624
ROWS_REM = N - NS * ROWS_PT
BLK = 1000


@functools.lru_cache(maxsize=None)
def _make_edge_agg(width):
    mesh = plsc.VectorSubcoreMesh(
        core_axis_name="c", subcore_axis_name="s",
        num_cores=NC, num_subcores=NS)

    @functools.partial(
        pl.kernel, mesh=mesh,
        out_type=jax.ShapeDtypeStruct((NC, N, width), jnp.float32),
        scratch_types=[
            pltpu.VMEM((HALF, CHUNK), jnp.int32),
            pltpu.VMEM((HALF, CHUNK), jnp.int32),
            pltpu.VMEM((2, CHUNK, width), jnp.float32),
            pltpu.VMEM_SHARED((NPAD, width), jnp.float32),
            pltpu.SemaphoreType.DMA,
            pltpu.SemaphoreType.DMA,
            pltpu.SemaphoreType.DMA,
            pltpu.SemaphoreType.DMA,
        ],
    )
    def k(feat_hbm, src_hbm, dst_hbm, out_hbm, src_v, dst_v, rows_v, acc_sh,
          semg0, semg1, sems0, sems1):
        c = lax.axis_index("c")
        s = lax.axis_index("s")
        wid = c * NS + s
        pltpu.sync_copy(feat_hbm.at[pl.ds(s * ROWS_PT, ROWS_PT)],
                        acc_sh.at[pl.ds(s * ROWS_PT, ROWS_PT)])

        @pl.when(s == NS - 1)
        def _():
            pltpu.sync_copy(feat_hbm.at[pl.ds(NS * ROWS_PT, ROWS_REM)],
                            acc_sh.at[pl.ds(NS * ROWS_PT, ROWS_REM)])

        plsc.subcore_barrier()

        semg = (semg0, semg1)
        sems = (sems0, sems1)
        for h in range(2):
            pltpu.sync_copy(src_hbm.at[wid, pl.ds(h * HALF, HALF)], src_v)
            pltpu.sync_copy(dst_hbm.at[wid, pl.ds(h * HALF, HALF)], dst_v)
            for b in range(2):
                pltpu.async_copy(feat_hbm.at[src_v.at[b]], rows_v.at[b],
                                 semg[b])

            @pl.loop(0, HALF, step=2)
            def _(j):
                for b in range(2):
                    pltpu.make_async_copy(feat_hbm.at[src_v.at[j + b]],
                                          rows_v.at[b], semg[b]).wait()
                    pltpu.async_copy(rows_v.at[b],
                                     acc_sh.at[dst_v.at[j + b]], sems[b],
                                     add=True)

                    @pl.when(j + b + 2 < HALF)
                    def _():
                        pltpu.make_async_copy(rows_v.at[b],
                                              acc_sh.at[dst_v.at[0]],
                                              sems[b]).wait()
                        pltpu.async_copy(feat_hbm.at[src_v.at[j + b + 2]],
                                         rows_v.at[b], semg[b])

            for b in range(2):
                pltpu.make_async_copy(rows_v.at[b], acc_sh.at[dst_v.at[0]],
                                      sems[b]).wait()

        plsc.subcore_barrier()
        pltpu.sync_copy(acc_sh.at[pl.ds(s * ROWS_PT, ROWS_PT)],
                        out_hbm.at[c, pl.ds(s * ROWS_PT, ROWS_PT)])

        @pl.when(s == NS - 1)
        def _():
            pltpu.sync_copy(acc_sh.at[pl.ds(NS * ROWS_PT, ROWS_REM)],
                            out_hbm.at[c, pl.ds(NS * ROWS_PT, ROWS_REM)])

    return k


def _edge_agg_d(feat, src, dst):
    return _make_edge_agg(D)(feat, src, dst)


@functools.lru_cache(maxsize=None)
def _make_scalar_agg():
    mesh = plsc.VectorSubcoreMesh(
        core_axis_name="c", subcore_axis_name="s",
        num_cores=NC, num_subcores=NS)

    @functools.partial(
        pl.kernel, mesh=mesh,
        out_type=jax.ShapeDtypeStruct((NC * N,), jnp.float32),
        scratch_types=[
            pltpu.VMEM((NCHUNK, CHUNK), jnp.int32),
            pltpu.VMEM((NCHUNK, CHUNK), jnp.int32),
            pltpu.VMEM((4, CHUNK), jnp.float32),
            pltpu.VMEM((ROWS_PT + ROWS_REM,), jnp.float32),
            pltpu.VMEM_SHARED((NPAD,), jnp.float32),
            pltpu.VMEM_SHARED((N,), jnp.float32),
            pltpu.SemaphoreType.DMA,
            pltpu.SemaphoreType.DMA,
            pltpu.SemaphoreType.DMA,
            pltpu.SemaphoreType.DMA,
            pltpu.SemaphoreType.DMA,
            pltpu.SemaphoreType.DMA,
            pltpu.SemaphoreType.DMA,
            pltpu.SemaphoreType.DMA,
        ],
    )
    def k(p_hbm, src_hbm, dst_hbm, out_hbm, src_v, dst_v, vals_v, stage_v,
          acc_sh, p_sh, semg0, semg1, semg2, semg3, sems0, sems1, sems2,
          sems3):
        c = lax.axis_index("c")
        s = lax.axis_index("s")
        wid = c * NS + s
        ibase = pl.multiple_of(s * ROWS_PT, 16)
        pltpu.sync_copy(p_hbm.at[pl.ds(ibase, ROWS_PT)],
                        stage_v.at[pl.ds(0, ROWS_PT)])
        pltpu.sync_copy(stage_v.at[pl.ds(0, ROWS_PT)],
                        acc_sh.at[pl.ds(ibase, ROWS_PT)])
        pltpu.sync_copy(stage_v.at[pl.ds(0, ROWS_PT)],
                        p_sh.at[pl.ds(ibase, ROWS_PT)])

        @pl.when(s == NS - 1)
        def _():
            pltpu.sync_copy(p_hbm.at[pl.ds(NS * ROWS_PT, ROWS_REM)],
                            stage_v.at[pl.ds(ROWS_PT, ROWS_REM)])
            pltpu.sync_copy(stage_v.at[pl.ds(ROWS_PT, ROWS_REM)],
                            acc_sh.at[pl.ds(NS * ROWS_PT, ROWS_REM)])
            pltpu.sync_copy(stage_v.at[pl.ds(ROWS_PT, ROWS_REM)],
                            p_sh.at[pl.ds(NS * ROWS_PT, ROWS_REM)])

        pltpu.sync_copy(src_hbm.at[wid], src_v)
        pltpu.sync_copy(dst_hbm.at[wid], dst_v)
        plsc.subcore_barrier()

        semg = (semg0, semg1, semg2, semg3)
        sems = (sems0, sems1, sems2, sems3)
        for b in range(4):
            pltpu.async_copy(p_sh.at[src_v.at[b]], vals_v.at[b], semg[b])

        @pl.loop(0, NCHUNK, step=4)
        def _(j):
            for b in range(4):
                pltpu.make_async_copy(p_sh.at[src_v.at[j + b]],
                                      vals_v.at[b], semg[b]).wait()
                pltpu.async_copy(vals_v.at[b], acc_sh.at[dst_v.at[j + b]],
                                 sems[b], add=True)

                @pl.when(j + b + 4 < NCHUNK)
                def _():
                    pltpu.make_async_copy(vals_v.at[b],
                                          acc_sh.at[dst_v.at[0]],
                                          sems[b]).wait()
                    pltpu.async_copy(p_sh.at[src_v.at[j + b + 4]],
                                     vals_v.at[b], semg[b])

        for b in range(4):
            pltpu.make_async_copy(vals_v.at[b], acc_sh.at[dst_v.at[0]],
                                  sems[b]).wait()

        plsc.subcore_barrier()
        obase = pl.multiple_of(c * N + s * ROWS_PT, 16)
        pltpu.sync_copy(acc_sh.at[pl.ds(ibase, ROWS_PT)],
                        stage_v.at[pl.ds(0, ROWS_PT)])
        pltpu.sync_copy(stage_v.at[pl.ds(0, ROWS_PT)],
                        out_hbm.at[pl.ds(obase, ROWS_PT)])

        @pl.when(s == NS - 1)
        def _():
            obase2 = pl.multiple_of(c * N + NS * ROWS_PT, 16)
            pltpu.sync_copy(acc_sh.at[pl.ds(NS * ROWS_PT, ROWS_REM)],
                            stage_v.at[pl.ds(ROWS_PT, ROWS_REM)])
            pltpu.sync_copy(stage_v.at[pl.ds(ROWS_PT, ROWS_REM)],
                            out_hbm.at[pl.ds(obase2, ROWS_REM)])

    return k


def _edge_agg_p(p_flat, src, dst):
    return _make_scalar_agg()(p_flat, src, dst)


def _split2(a):
    hi = a.astype(jnp.bfloat16)
    lo = (a - hi.astype(jnp.float32)).astype(jnp.bfloat16)
    return hi, lo


def _dot2(a, b):
    ah, al = _split2(a)
    bh, bl = _split2(b)
    d = functools.partial(jnp.dot, preferred_element_type=jnp.float32)
    return d(ah, bh) + (d(ah, bl) + d(al, bh))


def _seg_dot(onehot_bf16, xh, xl):
    dg = functools.partial(lax.dot_general,
                           dimension_numbers=(((0,), (0,)), ((), ())),
                           preferred_element_type=jnp.float32)
    return dg(onehot_bf16, xh) + dg(onehot_bf16, xl)


def _mlp_body(x_ref, a0_ref, a1_ref, W1_ref, b1_ref, W2_ref,
              b2_ref, W3t_ref, x2_ref, p_ref):
    xb = x_ref[...]
    hb = a0_ref[0] + a1_ref[0] - xb
    h1 = jnp.maximum(_dot2(hb, W1_ref[...]) + b1_ref[...], 0.0)
    x2 = jnp.maximum(_dot2(h1, W2_ref[...]) + b2_ref[...], 0.0)
    x2_ref[...] = x2
    wh, wl = _split2(W3t_ref[...])
    xh, xl = _split2(x2)
    dg = functools.partial(lax.dot_general,
                           dimension_numbers=(((1,), (1,)), ((), ())),
                           preferred_element_type=jnp.float32)
    p_ref[...] = (dg(wh, xh) + (dg(wh, xl) + dg(wl, xh)))[None]


def _out_body(x_ref, p_ref, q0_ref, q1_ref, batch_ref, scal_ref,
              x2o_ref, m1_ref, m2_ref):
    i = pl.program_id(0)
    p = p_ref[0]
    t = q0_ref[0] + q1_ref[0] - p
    b3 = scal_ref[0, 0]
    w4 = scal_ref[0, 1]
    b4 = scal_ref[0, 2]
    h2 = jnp.maximum(t + b3, 0.0)
    z = h2 * w4 + b4
    sig = 1.0 / (1.0 + jnp.exp(-z))
    xb = x_ref[...]
    x2o = sig.reshape(BLK, 1) * xb
    x2o_ref[...] = x2o
    onehot = (batch_ref[...] ==
              lax.broadcasted_iota(jnp.int32, (BLK, G), 1)
              ).astype(jnp.bfloat16)
    xh, xl = _split2(xb)
    m1_part = _seg_dot(onehot, xh, xl)
    oh, ol = _split2(x2o)
    m2_part = _seg_dot(onehot, oh, ol)

    @pl.when(i == 0)
    def _():
        m1_ref[...] = m1_part
        m2_ref[...] = m2_part

    @pl.when(i != 0)
    def _():
        m1_ref[...] += m1_part
        m2_ref[...] += m2_part


def _mlp_call(x, agg, W1, b1r, W2, b2r, W3t):
    grid = (N // BLK,)
    return pl.pallas_call(
        _mlp_body,
        grid=grid,
        in_specs=[
            pl.BlockSpec((BLK, D), lambda i: (i, 0)),
            pl.BlockSpec((1, BLK, D), lambda i: (0, i, 0)),
            pl.BlockSpec((1, BLK, D), lambda i: (1, i, 0)),
            pl.BlockSpec((D, H), lambda i: (0, 0)),
            pl.BlockSpec((1, H), lambda i: (0, 0)),
            pl.BlockSpec((H, H), lambda i: (0, 0)),
            pl.BlockSpec((1, H), lambda i: (0, 0)),
            pl.BlockSpec((1, H), lambda i: (0, 0)),
        ],
        out_specs=[
            pl.BlockSpec((BLK, D), lambda i: (i, 0)),
            pl.BlockSpec((1, 1, BLK), lambda i: (i, 0, 0)),
        ],
        out_shape=[
            jax.ShapeDtypeStruct((N, D), jnp.float32),
            jax.ShapeDtypeStruct((NB, 1, BLK), jnp.float32),
        ],
    )(x, agg, agg, W1, b1r, W2, b2r, W3t)


def _out_call(x, p, q, batch2d, scal):
    grid = (N // BLK,)
    return pl.pallas_call(
        _out_body,
        grid=grid,
        in_specs=[
            pl.BlockSpec((BLK, D), lambda i: (i, 0)),
            pl.BlockSpec((1, 1, BLK), lambda i: (i, 0, 0)),
            pl.BlockSpec((1, 1, BLK), lambda i: (i, 0, 0)),
            pl.BlockSpec((1, 1, BLK), lambda i: (NB + i, 0, 0)),
            pl.BlockSpec((BLK, 1), lambda i: (i, 0)),
            pl.BlockSpec(memory_space=pltpu.MemorySpace.SMEM),
        ],
        out_specs=[
            pl.BlockSpec((BLK, D), lambda i: (i, 0)),
            pl.BlockSpec((G, D), lambda i: (0, 0)),
            pl.BlockSpec((G, D), lambda i: (0, 0)),
        ],
        out_shape=[
            jax.ShapeDtypeStruct((N, D), jnp.float32),
            jax.ShapeDtypeStruct((G, D), jnp.float32),
            jax.ShapeDtypeStruct((G, D), jnp.float32),
        ],
    )(x, p, q, q, batch2d, scal)


def _prep_body(e_ref, out_ref):
    i = pl.program_id(0)
    pad = lax.broadcasted_iota(jnp.int32, (1, EPAD), 1)
    padv = jnp.where(i < NW, pad % N, N + (pad % 16))
    out_ref[0, 0, pl.ds(0, EPW)] = e_ref[0, 0]
    out_ref[0, 0, pl.ds(EPW, EPAD)] = padv[0]


def _prep_call(edges):
    return pl.pallas_call(
        _prep_body,
        grid=(2 * NW,),
        in_specs=[pl.BlockSpec((1, 1, EPW), lambda i: (i, 0, 0))],
        out_specs=pl.BlockSpec((1, 1, NCHUNK * CHUNK), lambda i: (i, 0, 0)),
        out_shape=jax.ShapeDtypeStruct((2 * NW, 1, NCHUNK * CHUNK),
                                       jnp.int32),
    )(edges)


def kernel(x, edge_index, batch, W1, b1, W2, b2, W3, b3, W4, b4):
    ed = _prep_call(
        edge_index.astype(jnp.int32).reshape(2 * NW, 1, EPW)
    ).reshape(2, NW, NCHUNK, CHUNK)
    src = ed[0]
    dst = ed[1]
    batch2d = batch.astype(jnp.int32).reshape(N, 1)
    agg = _edge_agg_d(x, src, dst)
    x2_hidden, p = _mlp_call(
        x, agg, W1, b1.reshape(1, H), W2, b2.reshape(1, H),
        W3.reshape(1, H))
    del x2_hidden
    q = _edge_agg_p(p.reshape(N), src, dst)
    q2 = q.reshape(NC * NB, 1, BLK)
    x2out, m1, m2 = _out_call(
        x, p, q2, batch2d,
        jnp.stack([b3[0], W4[0, 0], b4[0]]).reshape(1, 3))
    return (m1, m2, x, x2out)


# speedup vs baseline: 1.1416x; 1.1416x over previous
import functools

import numpy as np
import jax
import jax.numpy as jnp
from jax import lax
from jax.experimental import pallas as pl
from jax.experimental.pallas import tpu as pltpu
from jax.experimental.pallas import tpu_sc as plsc

N, D, H, E, G = 10000, 128, 128, 320000, 128
NB = 10
NC, NS = 2, 16
NW = NC * NS
EPW = E // NW
CHUNK = 128
NCHUNK = 80
HALF = NCHUNK // 2
EPAD = NCHUNK * CHUNK - EPW
NPAD = N + 16
DUMMY = N + 8
ROWS_PT = 624
ROWS_REM = N - NS * ROWS_PT
BLK = 1000


@functools.lru_cache(maxsize=None)
def _make_edge_agg(width):
    mesh = plsc.VectorSubcoreMesh(
        core_axis_name="c", subcore_axis_name="s",
        num_cores=NC, num_subcores=NS)

    @functools.partial(
        pl.kernel, mesh=mesh,
        out_type=jax.ShapeDtypeStruct((NC, N, width), jnp.float32),
        scratch_types=[
            pltpu.VMEM((HALF, CHUNK), jnp.int32),
            pltpu.VMEM((HALF, CHUNK), jnp.int32),
            pltpu.VMEM((2, CHUNK, width), jnp.float32),
            pltpu.VMEM_SHARED((NPAD, width), jnp.float32),
            pltpu.SemaphoreType.DMA,
            pltpu.SemaphoreType.DMA,
            pltpu.SemaphoreType.DMA,
            pltpu.SemaphoreType.DMA,
        ],
    )
    def k(feat_hbm, src_hbm, dst_hbm, out_hbm, src_v, dst_v, rows_v, acc_sh,
          semg0, semg1, sems0, sems1):
        c = lax.axis_index("c")
        s = lax.axis_index("s")
        wid = c * NS + s
        pltpu.sync_copy(feat_hbm.at[pl.ds(s * ROWS_PT, ROWS_PT)],
                        acc_sh.at[pl.ds(s * ROWS_PT, ROWS_PT)])

        @pl.when(s == NS - 1)
        def _():
            pltpu.sync_copy(feat_hbm.at[pl.ds(NS * ROWS_PT, ROWS_REM)],
                            acc_sh.at[pl.ds(NS * ROWS_PT, ROWS_REM)])

        plsc.subcore_barrier()

        semg = (semg0, semg1)
        sems = (sems0, sems1)
        for h in range(2):
            pltpu.sync_copy(src_hbm.at[wid, pl.ds(h * HALF, HALF)], src_v)
            pltpu.sync_copy(dst_hbm.at[wid, pl.ds(h * HALF, HALF)], dst_v)
            for b in range(2):
                pltpu.async_copy(feat_hbm.at[src_v.at[b]], rows_v.at[b],
                                 semg[b])

            @pl.loop(0, HALF, step=2)
            def _(j):
                for b in range(2):
                    pltpu.make_async_copy(feat_hbm.at[src_v.at[j + b]],
                                          rows_v.at[b], semg[b]).wait()
                    pltpu.async_copy(rows_v.at[b],
                                     acc_sh.at[dst_v.at[j + b]], sems[b],
                                     add=True)

                    @pl.when(j + b + 2 < HALF)
                    def _():
                        pltpu.make_async_copy(rows_v.at[b],
                                              acc_sh.at[dst_v.at[0]],
                                              sems[b]).wait()
                        pltpu.async_copy(feat_hbm.at[src_v.at[j + b + 2]],
                                         rows_v.at[b], semg[b])

            for b in range(2):
                pltpu.make_async_copy(rows_v.at[b], acc_sh.at[dst_v.at[0]],
                                      sems[b]).wait()

        plsc.subcore_barrier()
        pltpu.sync_copy(acc_sh.at[pl.ds(s * ROWS_PT, ROWS_PT)],
                        out_hbm.at[c, pl.ds(s * ROWS_PT, ROWS_PT)])

        @pl.when(s == NS - 1)
        def _():
            pltpu.sync_copy(acc_sh.at[pl.ds(NS * ROWS_PT, ROWS_REM)],
                            out_hbm.at[c, pl.ds(NS * ROWS_PT, ROWS_REM)])

    return k


def _edge_agg_d(feat, src, dst):
    return _make_edge_agg(D)(feat, src, dst)


@functools.lru_cache(maxsize=None)
def _make_scalar_agg():
    mesh = plsc.VectorSubcoreMesh(
        core_axis_name="c", subcore_axis_name="s",
        num_cores=NC, num_subcores=NS)

    @functools.partial(
        pl.kernel, mesh=mesh,
        out_type=jax.ShapeDtypeStruct((NC * N,), jnp.float32),
        scratch_types=[
            pltpu.VMEM((NCHUNK, CHUNK), jnp.int32),
            pltpu.VMEM((NCHUNK, CHUNK), jnp.int32),
            pltpu.VMEM((4, CHUNK), jnp.float32),
            pltpu.VMEM((ROWS_PT + ROWS_REM,), jnp.float32),
            pltpu.VMEM_SHARED((NPAD,), jnp.float32),
            pltpu.VMEM_SHARED((N,), jnp.float32),
            pltpu.SemaphoreType.DMA,
            pltpu.SemaphoreType.DMA,
            pltpu.SemaphoreType.DMA,
            pltpu.SemaphoreType.DMA,
            pltpu.SemaphoreType.DMA,
            pltpu.SemaphoreType.DMA,
            pltpu.SemaphoreType.DMA,
            pltpu.SemaphoreType.DMA,
        ],
    )
    def k(p_hbm, src_hbm, dst_hbm, out_hbm, src_v, dst_v, vals_v, stage_v,
          acc_sh, p_sh, semg0, semg1, semg2, semg3, sems0, sems1, sems2,
          sems3):
        c = lax.axis_index("c")
        s = lax.axis_index("s")
        wid = c * NS + s
        ibase = pl.multiple_of(s * ROWS_PT, 16)
        pltpu.sync_copy(p_hbm.at[pl.ds(ibase, ROWS_PT)],
                        stage_v.at[pl.ds(0, ROWS_PT)])
        pltpu.sync_copy(stage_v.at[pl.ds(0, ROWS_PT)],
                        acc_sh.at[pl.ds(ibase, ROWS_PT)])
        pltpu.sync_copy(stage_v.at[pl.ds(0, ROWS_PT)],
                        p_sh.at[pl.ds(ibase, ROWS_PT)])

        @pl.when(s == NS - 1)
        def _():
            pltpu.sync_copy(p_hbm.at[pl.ds(NS * ROWS_PT, ROWS_REM)],
                            stage_v.at[pl.ds(ROWS_PT, ROWS_REM)])
            pltpu.sync_copy(stage_v.at[pl.ds(ROWS_PT, ROWS_REM)],
                            acc_sh.at[pl.ds(NS * ROWS_PT, ROWS_REM)])
            pltpu.sync_copy(stage_v.at[pl.ds(ROWS_PT, ROWS_REM)],
                            p_sh.at[pl.ds(NS * ROWS_PT, ROWS_REM)])

        pltpu.sync_copy(src_hbm.at[wid], src_v)
        pltpu.sync_copy(dst_hbm.at[wid], dst_v)
        plsc.subcore_barrier()

        semg = (semg0, semg1, semg2, semg3)
        sems = (sems0, sems1, sems2, sems3)
        for b in range(4):
            pltpu.async_copy(p_sh.at[src_v.at[b]], vals_v.at[b], semg[b])

        @pl.loop(0, NCHUNK, step=4)
        def _(j):
            for b in range(4):
                pltpu.make_async_copy(p_sh.at[src_v.at[j + b]],
                                      vals_v.at[b], semg[b]).wait()
                pltpu.async_copy(vals_v.at[b], acc_sh.at[dst_v.at[j + b]],
                                 sems[b], add=True)

                @pl.when(j + b + 4 < NCHUNK)
                def _():
                    pltpu.make_async_copy(vals_v.at[b],
                                          acc_sh.at[dst_v.at[0]],
                                          sems[b]).wait()
                    pltpu.async_copy(p_sh.at[src_v.at[j + b + 4]],
                                     vals_v.at[b], semg[b])

        for b in range(4):
            pltpu.make_async_copy(vals_v.at[b], acc_sh.at[dst_v.at[0]],
                                  sems[b]).wait()

        plsc.subcore_barrier()
        obase = pl.multiple_of(c * N + s * ROWS_PT, 16)
        pltpu.sync_copy(acc_sh.at[pl.ds(ibase, ROWS_PT)],
                        stage_v.at[pl.ds(0, ROWS_PT)])
        pltpu.sync_copy(stage_v.at[pl.ds(0, ROWS_PT)],
                        out_hbm.at[pl.ds(obase, ROWS_PT)])

        @pl.when(s == NS - 1)
        def _():
            obase2 = pl.multiple_of(c * N + NS * ROWS_PT, 16)
            pltpu.sync_copy(acc_sh.at[pl.ds(NS * ROWS_PT, ROWS_REM)],
                            stage_v.at[pl.ds(ROWS_PT, ROWS_REM)])
            pltpu.sync_copy(stage_v.at[pl.ds(ROWS_PT, ROWS_REM)],
                            out_hbm.at[pl.ds(obase2, ROWS_REM)])

    return k


def _edge_agg_p(p_flat, src, dst):
    return _make_scalar_agg()(p_flat, src, dst)


def _split2(a):
    hi = a.astype(jnp.bfloat16)
    lo = (a - hi.astype(jnp.float32)).astype(jnp.bfloat16)
    return hi, lo


def _dot2(a, b):
    ah, al = _split2(a)
    bh, bl = _split2(b)
    d = functools.partial(jnp.dot, preferred_element_type=jnp.float32)
    return d(ah, bh) + (d(ah, bl) + d(al, bh))


def _seg_dot(onehot_bf16, xh, xl):
    dg = functools.partial(lax.dot_general,
                           dimension_numbers=(((0,), (0,)), ((), ())),
                           preferred_element_type=jnp.float32)
    return dg(onehot_bf16, xh) + dg(onehot_bf16, xl)


def _mlp_body(x_ref, a0_ref, a1_ref, W1_ref, b1_ref, W2_ref,
              b2_ref, W3t_ref, x2_ref, p_ref):
    xb = x_ref[...]
    hb = a0_ref[0] + a1_ref[0] - xb
    h1 = jnp.maximum(_dot2(hb, W1_ref[...]) + b1_ref[...], 0.0)
    x2 = jnp.maximum(_dot2(h1, W2_ref[...]) + b2_ref[...], 0.0)
    x2_ref[...] = x2
    wh, wl = _split2(W3t_ref[...])
    xh, xl = _split2(x2)
    dg = functools.partial(lax.dot_general,
                           dimension_numbers=(((1,), (1,)), ((), ())),
                           preferred_element_type=jnp.float32)
    p_ref[...] = (dg(wh, xh) + (dg(wh, xl) + dg(wl, xh)))[None]


def _out_body(x_ref, p_ref, q0_ref, q1_ref, batch_ref, scal_ref,
              x2o_ref, m1_ref, m2_ref):
    i = pl.program_id(0)
    p = p_ref[0]
    t = q0_ref[0] + q1_ref[0] - p
    b3 = scal_ref[0, 0]
    w4 = scal_ref[0, 1]
    b4 = scal_ref[0, 2]
    h2 = jnp.maximum(t + b3, 0.0)
    z = h2 * w4 + b4
    sig = 1.0 / (1.0 + jnp.exp(-z))
    xb = x_ref[...]
    x2o = sig.reshape(BLK, 1) * xb
    x2o_ref[...] = x2o
    onehot = (batch_ref[...] ==
              lax.broadcasted_iota(jnp.int32, (BLK, G), 1)
              ).astype(jnp.bfloat16)
    xh, xl = _split2(xb)
    m1_part = _seg_dot(onehot, xh, xl)
    oh, ol = _split2(x2o)
    m2_part = _seg_dot(onehot, oh, ol)

    @pl.when(i == 0)
    def _():
        m1_ref[...] = m1_part
        m2_ref[...] = m2_part

    @pl.when(i != 0)
    def _():
        m1_ref[...] += m1_part
        m2_ref[...] += m2_part


def _mlp_call(x, agg, W1, b1r, W2, b2r, W3t):
    grid = (N // BLK,)
    return pl.pallas_call(
        _mlp_body,
        grid=grid,
        in_specs=[
            pl.BlockSpec((BLK, D), lambda i: (i, 0)),
            pl.BlockSpec((1, BLK, D), lambda i: (0, i, 0)),
            pl.BlockSpec((1, BLK, D), lambda i: (1, i, 0)),
            pl.BlockSpec((D, H), lambda i: (0, 0)),
            pl.BlockSpec((1, H), lambda i: (0, 0)),
            pl.BlockSpec((H, H), lambda i: (0, 0)),
            pl.BlockSpec((1, H), lambda i: (0, 0)),
            pl.BlockSpec((1, H), lambda i: (0, 0)),
        ],
        out_specs=[
            pl.BlockSpec((BLK, D), lambda i: (i, 0)),
            pl.BlockSpec((1, 1, BLK), lambda i: (i, 0, 0)),
        ],
        out_shape=[
            jax.ShapeDtypeStruct((N, D), jnp.float32),
            jax.ShapeDtypeStruct((NB, 1, BLK), jnp.float32),
        ],
    )(x, agg, agg, W1, b1r, W2, b2r, W3t)


def _out_call(x, p, q, batch2d, scal):
    grid = (N // BLK,)
    return pl.pallas_call(
        _out_body,
        grid=grid,
        in_specs=[
            pl.BlockSpec((BLK, D), lambda i: (i, 0)),
            pl.BlockSpec((1, 1, BLK), lambda i: (i, 0, 0)),
            pl.BlockSpec((1, 1, BLK), lambda i: (i, 0, 0)),
            pl.BlockSpec((1, 1, BLK), lambda i: (NB + i, 0, 0)),
            pl.BlockSpec((BLK, 1), lambda i: (i, 0)),
            pl.BlockSpec(memory_space=pltpu.MemorySpace.SMEM),
        ],
        out_specs=[
            pl.BlockSpec((BLK, D), lambda i: (i, 0)),
            pl.BlockSpec((G, D), lambda i: (0, 0)),
            pl.BlockSpec((G, D), lambda i: (0, 0)),
        ],
        out_shape=[
            jax.ShapeDtypeStruct((N, D), jnp.float32),
            jax.ShapeDtypeStruct((G, D), jnp.float32),
            jax.ShapeDtypeStruct((G, D), jnp.float32),
        ],
    )(x, p, q, q, batch2d, scal)


_PAD_SRC = np.broadcast_to(np.arange(EPAD, dtype=np.int32) % N,
                           (NW, EPAD))
_PAD_DST = np.broadcast_to(N + (np.arange(EPAD, dtype=np.int32) % 16),
                           (NW, EPAD))


def kernel(x, edge_index, batch, W1, b1, W2, b2, W3, b3, W4, b4):
    src = jnp.concatenate(
        [edge_index[0].astype(jnp.int32).reshape(NW, EPW),
         jnp.asarray(_PAD_SRC)], axis=1).reshape(NW, NCHUNK, CHUNK)
    dst = jnp.concatenate(
        [edge_index[1].astype(jnp.int32).reshape(NW, EPW),
         jnp.asarray(_PAD_DST)], axis=1).reshape(NW, NCHUNK, CHUNK)
    batch2d = batch.astype(jnp.int32).reshape(N, 1)
    agg = _edge_agg_d(x, src, dst)
    x2_hidden, p = _mlp_call(
        x, agg, W1, b1.reshape(1, H), W2, b2.reshape(1, H),
        W3.reshape(1, H))
    del x2_hidden
    q = _edge_agg_p(p.reshape(N), src, dst)
    q2 = q.reshape(NC * NB, 1, BLK)
    x2out, m1, m2 = _out_call(
        x, p, q2, batch2d,
        jnp.stack([b3[0], W4[0, 0], b4[0]]).reshape(1, 3))
    return (m1, m2, x, x2out)


# speedup vs baseline: 1.1684x; 1.0234x over previous
import functools

import numpy as np
import jax
import jax.numpy as jnp
from jax import lax
from jax.experimental import pallas as pl
from jax.experimental.pallas import tpu as pltpu
from jax.experimental.pallas import tpu_sc as plsc

N, D, H, E, G = 10000, 128, 128, 320000, 128
NB = 5
NC, NS = 2, 16
NW = NC * NS
EPW = E // NW
CHUNK = 128
NCHUNK = 80
HALF = NCHUNK // 2
EPAD = NCHUNK * CHUNK - EPW
NPAD = N + 16
DUMMY = N + 8
ROWS_PT = 624
ROWS_REM = N - NS * ROWS_PT
BLK = 2000


@functools.lru_cache(maxsize=None)
def _make_edge_agg(width):
    mesh = plsc.VectorSubcoreMesh(
        core_axis_name="c", subcore_axis_name="s",
        num_cores=NC, num_subcores=NS)

    @functools.partial(
        pl.kernel, mesh=mesh,
        out_type=jax.ShapeDtypeStruct((NC, N, width), jnp.float32),
        scratch_types=[
            pltpu.VMEM((HALF, CHUNK), jnp.int32),
            pltpu.VMEM((HALF, CHUNK), jnp.int32),
            pltpu.VMEM((2, CHUNK, width), jnp.float32),
            pltpu.VMEM_SHARED((NPAD, width), jnp.float32),
            pltpu.SemaphoreType.DMA,
            pltpu.SemaphoreType.DMA,
            pltpu.SemaphoreType.DMA,
            pltpu.SemaphoreType.DMA,
        ],
    )
    def k(feat_hbm, src_hbm, dst_hbm, out_hbm, src_v, dst_v, rows_v, acc_sh,
          semg0, semg1, sems0, sems1):
        c = lax.axis_index("c")
        s = lax.axis_index("s")
        wid = c * NS + s
        pltpu.sync_copy(feat_hbm.at[pl.ds(s * ROWS_PT, ROWS_PT)],
                        acc_sh.at[pl.ds(s * ROWS_PT, ROWS_PT)])

        @pl.when(s == NS - 1)
        def _():
            pltpu.sync_copy(feat_hbm.at[pl.ds(NS * ROWS_PT, ROWS_REM)],
                            acc_sh.at[pl.ds(NS * ROWS_PT, ROWS_REM)])

        plsc.subcore_barrier()

        semg = (semg0, semg1)
        sems = (sems0, sems1)
        for h in range(2):
            pltpu.sync_copy(src_hbm.at[wid, pl.ds(h * HALF, HALF)], src_v)
            pltpu.sync_copy(dst_hbm.at[wid, pl.ds(h * HALF, HALF)], dst_v)
            for b in range(2):
                pltpu.async_copy(feat_hbm.at[src_v.at[b]], rows_v.at[b],
                                 semg[b])

            @pl.loop(0, HALF, step=2)
            def _(j):
                for b in range(2):
                    pltpu.make_async_copy(feat_hbm.at[src_v.at[j + b]],
                                          rows_v.at[b], semg[b]).wait()
                    pltpu.async_copy(rows_v.at[b],
                                     acc_sh.at[dst_v.at[j + b]], sems[b],
                                     add=True)

                    @pl.when(j + b + 2 < HALF)
                    def _():
                        pltpu.make_async_copy(rows_v.at[b],
                                              acc_sh.at[dst_v.at[0]],
                                              sems[b]).wait()
                        pltpu.async_copy(feat_hbm.at[src_v.at[j + b + 2]],
                                         rows_v.at[b], semg[b])

            for b in range(2):
                pltpu.make_async_copy(rows_v.at[b], acc_sh.at[dst_v.at[0]],
                                      sems[b]).wait()

        plsc.subcore_barrier()
        pltpu.sync_copy(acc_sh.at[pl.ds(s * ROWS_PT, ROWS_PT)],
                        out_hbm.at[c, pl.ds(s * ROWS_PT, ROWS_PT)])

        @pl.when(s == NS - 1)
        def _():
            pltpu.sync_copy(acc_sh.at[pl.ds(NS * ROWS_PT, ROWS_REM)],
                            out_hbm.at[c, pl.ds(NS * ROWS_PT, ROWS_REM)])

    return k


def _edge_agg_d(feat, src, dst):
    return _make_edge_agg(D)(feat, src, dst)


@functools.lru_cache(maxsize=None)
def _make_scalar_agg():
    mesh = plsc.VectorSubcoreMesh(
        core_axis_name="c", subcore_axis_name="s",
        num_cores=NC, num_subcores=NS)

    @functools.partial(
        pl.kernel, mesh=mesh,
        out_type=jax.ShapeDtypeStruct((NC * N,), jnp.float32),
        scratch_types=[
            pltpu.VMEM((NCHUNK, CHUNK), jnp.int32),
            pltpu.VMEM((NCHUNK, CHUNK), jnp.int32),
            pltpu.VMEM((4, CHUNK), jnp.float32),
            pltpu.VMEM((ROWS_PT + ROWS_REM,), jnp.float32),
            pltpu.VMEM_SHARED((NPAD,), jnp.float32),
            pltpu.VMEM_SHARED((N,), jnp.float32),
            pltpu.SemaphoreType.DMA,
            pltpu.SemaphoreType.DMA,
            pltpu.SemaphoreType.DMA,
            pltpu.SemaphoreType.DMA,
            pltpu.SemaphoreType.DMA,
            pltpu.SemaphoreType.DMA,
            pltpu.SemaphoreType.DMA,
            pltpu.SemaphoreType.DMA,
        ],
    )
    def k(p_hbm, src_hbm, dst_hbm, out_hbm, src_v, dst_v, vals_v, stage_v,
          acc_sh, p_sh, semg0, semg1, semg2, semg3, sems0, sems1, sems2,
          sems3):
        c = lax.axis_index("c")
        s = lax.axis_index("s")
        wid = c * NS + s
        ibase = pl.multiple_of(s * ROWS_PT, 16)
        pltpu.sync_copy(p_hbm.at[pl.ds(ibase, ROWS_PT)],
                        stage_v.at[pl.ds(0, ROWS_PT)])
        pltpu.sync_copy(stage_v.at[pl.ds(0, ROWS_PT)],
                        acc_sh.at[pl.ds(ibase, ROWS_PT)])
        pltpu.sync_copy(stage_v.at[pl.ds(0, ROWS_PT)],
                        p_sh.at[pl.ds(ibase, ROWS_PT)])

        @pl.when(s == NS - 1)
        def _():
            pltpu.sync_copy(p_hbm.at[pl.ds(NS * ROWS_PT, ROWS_REM)],
                            stage_v.at[pl.ds(ROWS_PT, ROWS_REM)])
            pltpu.sync_copy(stage_v.at[pl.ds(ROWS_PT, ROWS_REM)],
                            acc_sh.at[pl.ds(NS * ROWS_PT, ROWS_REM)])
            pltpu.sync_copy(stage_v.at[pl.ds(ROWS_PT, ROWS_REM)],
                            p_sh.at[pl.ds(NS * ROWS_PT, ROWS_REM)])

        pltpu.sync_copy(src_hbm.at[wid], src_v)
        pltpu.sync_copy(dst_hbm.at[wid], dst_v)
        plsc.subcore_barrier()

        semg = (semg0, semg1, semg2, semg3)
        sems = (sems0, sems1, sems2, sems3)
        for b in range(4):
            pltpu.async_copy(p_sh.at[src_v.at[b]], vals_v.at[b], semg[b])

        @pl.loop(0, NCHUNK, step=4)
        def _(j):
            for b in range(4):
                pltpu.make_async_copy(p_sh.at[src_v.at[j + b]],
                                      vals_v.at[b], semg[b]).wait()
                pltpu.async_copy(vals_v.at[b], acc_sh.at[dst_v.at[j + b]],
                                 sems[b], add=True)

                @pl.when(j + b + 4 < NCHUNK)
                def _():
                    pltpu.make_async_copy(vals_v.at[b],
                                          acc_sh.at[dst_v.at[0]],
                                          sems[b]).wait()
                    pltpu.async_copy(p_sh.at[src_v.at[j + b + 4]],
                                     vals_v.at[b], semg[b])

        for b in range(4):
            pltpu.make_async_copy(vals_v.at[b], acc_sh.at[dst_v.at[0]],
                                  sems[b]).wait()

        plsc.subcore_barrier()
        obase = pl.multiple_of(c * N + s * ROWS_PT, 16)
        pltpu.sync_copy(acc_sh.at[pl.ds(ibase, ROWS_PT)],
                        stage_v.at[pl.ds(0, ROWS_PT)])
        pltpu.sync_copy(stage_v.at[pl.ds(0, ROWS_PT)],
                        out_hbm.at[pl.ds(obase, ROWS_PT)])

        @pl.when(s == NS - 1)
        def _():
            obase2 = pl.multiple_of(c * N + NS * ROWS_PT, 16)
            pltpu.sync_copy(acc_sh.at[pl.ds(NS * ROWS_PT, ROWS_REM)],
                            stage_v.at[pl.ds(ROWS_PT, ROWS_REM)])
            pltpu.sync_copy(stage_v.at[pl.ds(ROWS_PT, ROWS_REM)],
                            out_hbm.at[pl.ds(obase2, ROWS_REM)])

    return k


def _edge_agg_p(p_flat, src, dst):
    return _make_scalar_agg()(p_flat, src, dst)


def _split2(a):
    hi = a.astype(jnp.bfloat16)
    lo = (a - hi.astype(jnp.float32)).astype(jnp.bfloat16)
    return hi, lo


def _dot2(a, b):
    ah, al = _split2(a)
    bh, bl = _split2(b)
    d = functools.partial(jnp.dot, preferred_element_type=jnp.float32)
    return d(ah, bh) + (d(ah, bl) + d(al, bh))


def _seg_dot(onehot_bf16, xh, xl):
    dg = functools.partial(lax.dot_general,
                           dimension_numbers=(((0,), (0,)), ((), ())),
                           preferred_element_type=jnp.float32)
    return dg(onehot_bf16, xh) + dg(onehot_bf16, xl)


def _mlp_body(x_ref, a0_ref, a1_ref, W1_ref, b1_ref, W2_ref,
              b2_ref, W3t_ref, x2_ref, p_ref):
    xb = x_ref[...]
    hb = a0_ref[0] + a1_ref[0] - xb
    h1 = jnp.maximum(_dot2(hb, W1_ref[...]) + b1_ref[...], 0.0)
    x2 = jnp.maximum(_dot2(h1, W2_ref[...]) + b2_ref[...], 0.0)
    x2_ref[...] = x2
    wh, wl = _split2(W3t_ref[...])
    xh, xl = _split2(x2)
    dg = functools.partial(lax.dot_general,
                           dimension_numbers=(((1,), (1,)), ((), ())),
                           preferred_element_type=jnp.float32)
    p_ref[...] = (dg(wh, xh) + (dg(wh, xl) + dg(wl, xh)))[None]


def _out_body(x_ref, p_ref, q0_ref, q1_ref, batch_ref, scal_ref,
              x2o_ref, m1_ref, m2_ref):
    i = pl.program_id(0)
    p = p_ref[0]
    t = q0_ref[0] + q1_ref[0] - p
    b3 = scal_ref[0, 0]
    w4 = scal_ref[0, 1]
    b4 = scal_ref[0, 2]
    h2 = jnp.maximum(t + b3, 0.0)
    z = h2 * w4 + b4
    sig = 1.0 / (1.0 + jnp.exp(-z))
    xb = x_ref[...]
    x2o = sig.reshape(BLK, 1) * xb
    x2o_ref[...] = x2o
    onehot = (batch_ref[...] ==
              lax.broadcasted_iota(jnp.int32, (BLK, G), 1)
              ).astype(jnp.bfloat16)
    xh, xl = _split2(xb)
    m1_part = _seg_dot(onehot, xh, xl)
    oh, ol = _split2(x2o)
    m2_part = _seg_dot(onehot, oh, ol)

    @pl.when(i == 0)
    def _():
        m1_ref[...] = m1_part
        m2_ref[...] = m2_part

    @pl.when(i != 0)
    def _():
        m1_ref[...] += m1_part
        m2_ref[...] += m2_part


def _mlp_call(x, agg, W1, b1r, W2, b2r, W3t):
    grid = (N // BLK,)
    return pl.pallas_call(
        _mlp_body,
        grid=grid,
        in_specs=[
            pl.BlockSpec((BLK, D), lambda i: (i, 0)),
            pl.BlockSpec((1, BLK, D), lambda i: (0, i, 0)),
            pl.BlockSpec((1, BLK, D), lambda i: (1, i, 0)),
            pl.BlockSpec((D, H), lambda i: (0, 0)),
            pl.BlockSpec((1, H), lambda i: (0, 0)),
            pl.BlockSpec((H, H), lambda i: (0, 0)),
            pl.BlockSpec((1, H), lambda i: (0, 0)),
            pl.BlockSpec((1, H), lambda i: (0, 0)),
        ],
        out_specs=[
            pl.BlockSpec((BLK, D), lambda i: (i, 0)),
            pl.BlockSpec((1, 1, BLK), lambda i: (i, 0, 0)),
        ],
        out_shape=[
            jax.ShapeDtypeStruct((N, D), jnp.float32),
            jax.ShapeDtypeStruct((NB, 1, BLK), jnp.float32),
        ],
    )(x, agg, agg, W1, b1r, W2, b2r, W3t)


def _out_call(x, p, q, batch2d, scal):
    grid = (N // BLK,)
    return pl.pallas_call(
        _out_body,
        grid=grid,
        in_specs=[
            pl.BlockSpec((BLK, D), lambda i: (i, 0)),
            pl.BlockSpec((1, 1, BLK), lambda i: (i, 0, 0)),
            pl.BlockSpec((1, 1, BLK), lambda i: (i, 0, 0)),
            pl.BlockSpec((1, 1, BLK), lambda i: (NB + i, 0, 0)),
            pl.BlockSpec((BLK, 1), lambda i: (i, 0)),
            pl.BlockSpec(memory_space=pltpu.MemorySpace.SMEM),
        ],
        out_specs=[
            pl.BlockSpec((BLK, D), lambda i: (i, 0)),
            pl.BlockSpec((G, D), lambda i: (0, 0)),
            pl.BlockSpec((G, D), lambda i: (0, 0)),
        ],
        out_shape=[
            jax.ShapeDtypeStruct((N, D), jnp.float32),
            jax.ShapeDtypeStruct((G, D), jnp.float32),
            jax.ShapeDtypeStruct((G, D), jnp.float32),
        ],
    )(x, p, q, q, batch2d, scal)


_PAD_SRC = np.broadcast_to(np.arange(EPAD, dtype=np.int32) % N,
                           (NW, EPAD))
_PAD_DST = np.broadcast_to(N + (np.arange(EPAD, dtype=np.int32) % 16),
                           (NW, EPAD))


def kernel(x, edge_index, batch, W1, b1, W2, b2, W3, b3, W4, b4):
    src = jnp.concatenate(
        [edge_index[0].astype(jnp.int32).reshape(NW, EPW),
         jnp.asarray(_PAD_SRC)], axis=1).reshape(NW, NCHUNK, CHUNK)
    dst = jnp.concatenate(
        [edge_index[1].astype(jnp.int32).reshape(NW, EPW),
         jnp.asarray(_PAD_DST)], axis=1).reshape(NW, NCHUNK, CHUNK)
    batch2d = batch.astype(jnp.int32).reshape(N, 1)
    agg = _edge_agg_d(x, src, dst)
    x2_hidden, p = _mlp_call(
        x, agg, W1, b1.reshape(1, H), W2, b2.reshape(1, H),
        W3.reshape(1, H))
    del x2_hidden
    q = _edge_agg_p(p.reshape(N), src, dst)
    q2 = q.reshape(NC * NB, 1, BLK)
    x2out, m1, m2 = _out_call(
        x, p, q2, batch2d,
        jnp.stack([b3[0], W4[0, 0], b4[0]]).reshape(1, 3))
    return (m1, m2, x, x2out)
